# baseline (speedup 1.0000x reference)
import jax
import jax.numpy as jnp
from jax import lax
from jax.experimental import pallas as pl
from jax.experimental.pallas import tpu as pltpu
from jax.experimental.pallas import tpu_sc as plsc

B = 64
H = 512
W = 512
N = H * W
K = 1 + round(0.01 * 99.0 * (N - 1))

NC = 2
NS = 16
NW = NC * NS
SPT = B // NW

NB = 4096
NB_C = 128
L = 16
CH = 8192
NCH = N // CH
VPC = CH // L
CAP = 32768


def _tc_body(pred_ref, tgt_ref, mask_ref, loss_ref, sums_ref):
    lv = mask_ref[...] * lax.abs(pred_ref[...] - tgt_ref[...])
    loss_ref[...] = lv
    sums_ref[...] = jnp.full((1, 1, 128), jnp.sum(lv), jnp.float32)


def _tc_loss(pred, tgt, mask):
    return pl.pallas_call(
        _tc_body,
        grid=(B,),
        in_specs=[
            pl.BlockSpec((1, H, W), lambda b: (b, 0, 0)),
            pl.BlockSpec((1, H, W), lambda b: (b, 0, 0)),
            pl.BlockSpec((1, H, W), lambda b: (b, 0, 0)),
        ],
        out_specs=[
            pl.BlockSpec((1, H, W), lambda b: (b, 0, 0)),
            pl.BlockSpec((1, 1, 128), lambda b: (b, 0, 0)),
        ],
        out_shape=[
            jax.ShapeDtypeStruct((B, H, W), jnp.float32),
            jax.ShapeDtypeStruct((B, 1, 128), jnp.float32),
        ],
    )(pred, tgt, mask)


def _zero_hist(hist, nb):
    def body(i, _):
        for bank in range(L):
            hist[pl.ds(bank * NB + i * L, L)] = jnp.zeros((L,), jnp.int32)
        return 0
    lax.fori_loop(0, nb // L, body, 0)


def _select(hist, tmp, tmp2, r, nb):
    lane = lax.broadcasted_iota(jnp.int32, (L,), 0)
    zvec = jnp.zeros((L,), jnp.int32)
    ones_v = jnp.ones((L,), jnp.int32)
    nblk = nb // L

    def p1(j, _):
        v = zvec
        for bank in range(L):
            v = v + hist[pl.ds(bank * NB + j * L, L)]
            hist[pl.ds(bank * NB + j * L, L)] = zvec
        tmp[pl.ds(j * L, L)] = v
        return 0

    lax.fori_loop(0, nblk, p1, 0)

    def p2a(j, _):
        t = tmp[pl.ds(j * L, L)]
        tot = jnp.sum(t)
        plsc.store_scatter(tmp2, [jnp.full((L,), j, jnp.int32)],
                           jnp.full((L,), tot, jnp.int32),
                           mask=lane == 0)
        return 0

    lax.fori_loop(0, nblk, p2a, 0)

    zero = jnp.int32(0)
    nv2 = (nblk + L - 1) // L

    def p2b(jj, carry):
        cum, bblk, cumbef = carry
        t = tmp2[pl.ds(jj * L, L)]
        t = jnp.where((jj * L + lane) < nblk, t, zvec)
        cv = plsc.cumsum(t) + cum
        mlt = cv < r
        bblk = bblk + jnp.sum(jnp.where(mlt, ones_v, zvec))
        cumbef = cumbef + jnp.sum(jnp.where(mlt, t, zvec))
        cum = cum + jnp.sum(t)
        return (cum, bblk, cumbef)

    _cum, jb, cumbef = lax.fori_loop(0, nv2, p2b, (zero, zero, zero))

    v = tmp[pl.ds(jb * L, L)]
    cv = plsc.cumsum(v) + cumbef
    mlt = cv < r
    msel = jnp.logical_and(cv >= r, (cv - v) < r)
    loff = jnp.sum(jnp.where(mlt, ones_v, zvec))
    cumbef = cumbef + jnp.sum(jnp.where(mlt, v, zvec))
    cnt = jnp.sum(jnp.where(msel, v, zvec))
    return jb * L + loff, r - cumbef, cnt


def _stream_pass(src_hbm, s, buf0, buf1, sem0, sem1, chunk_fn, init_carry):
    def start(c, buf, sem):
        off = pl.multiple_of(c * CH, CH)
        pltpu.async_copy(src_hbm.at[s, pl.ds(off, CH)], buf, sem)

    def wait(c, buf, sem):
        off = pl.multiple_of(c * CH, CH)
        pltpu.make_async_copy(src_hbm.at[s, pl.ds(off, CH)], buf, sem).wait()

    start(0, buf0, sem0)
    start(1, buf1, sem1)

    def body(c2, carry):
        c0 = c2 * 2
        wait(c0, buf0, sem0)
        carry = chunk_fn(buf0, c0, carry)

        @pl.when(c0 + 2 < NCH)
        def _():
            start(c0 + 2, buf0, sem0)

        wait(c0 + 1, buf1, sem1)
        carry = chunk_fn(buf1, c0 + 1, carry)

        @pl.when(c0 + 3 < NCH)
        def _():
            start(c0 + 3, buf1, sem1)

        return carry

    return lax.fori_loop(0, NCH // 2, body, init_carry)


def _sc_body(loss_hbm, qbits_hbm,
             buf0, buf1, cand, hist, tmp, tmp2, outbuf_i, sem0, sem1):
    wid = lax.axis_index("s") * NC + lax.axis_index("c")
    lane = lax.broadcasted_iota(jnp.int32, (L,), 0)
    ones_i = jnp.ones((L,), jnp.int32)

    _zero_hist(hist, NB)

    results = []
    for local in range(SPT):
        s = wid * SPT + local


        def chunk_a(buf, c, carry):
            def vbody(i, _):
                bits = lax.bitcast_convert_type(buf[pl.ds(i * L, L)],
                                                jnp.int32)
                d = lax.shift_right_logical(bits, 19)
                plsc.addupdate_scatter(hist, [lane * NB + d], ones_i)
                return 0
            lax.fori_loop(0, VPC, vbody, 0)
            return carry

        _stream_pass(loss_hbm, s, buf0, buf1, sem0, sem1, chunk_a, 0)
        b1, r2, cnt1 = _select(hist, tmp, tmp2, jnp.int32(K), NB)
        docap = cnt1 <= CAP

        def chunk_b(buf, c, cnt_splat):
            def vbody(i, cnt_splat):
                bits = lax.bitcast_convert_type(buf[pl.ds(i * L, L)],
                                                jnp.int32)
                match = lax.shift_right_logical(bits, 19) == b1
                d = lax.bitwise_and(lax.shift_right_logical(bits, 7),
                                    jnp.int32(0xFFF))
                plsc.addupdate_scatter(hist, [lane * NB + d], ones_i,
                                       mask=match)
                mi = match.astype(jnp.int32)
                pos = cnt_splat + plsc.cumsum(mi) - 1
                plsc.store_scatter(cand, [pos], bits,
                                   mask=jnp.logical_and(match, docap))
                return cnt_splat + plsc.all_reduce_population_count(match)
            return lax.fori_loop(0, VPC, vbody, cnt_splat)

        _stream_pass(loss_hbm, s, buf0, buf1, sem0, sem1, chunk_b,
                     jnp.zeros((L,), jnp.int32))
        b2, r3, _cnt2 = _select(hist, tmp, tmp2, r2, NB)
        prefix24 = b1 * 4096 + b2

        @pl.when(docap)
        def _():
            nv = (cnt1 + (L - 1)) // L

            def vbody(i, _):
                bits = cand[pl.ds(i * L, L)]
                inb = (i * L + lane) < cnt1
                match = jnp.logical_and(
                    lax.shift_right_logical(bits, 7) == prefix24, inb)
                d = lax.bitwise_and(bits, jnp.int32(0x7F))
                plsc.addupdate_scatter(hist, [lane * NB + d], ones_i,
                                       mask=match)
                return 0

            lax.fori_loop(0, nv, vbody, 0)

        @pl.when(jnp.logical_not(docap))
        def _():
            def chunk_c(buf, c, carry):
                def vbody(i, _):
                    bits = lax.bitcast_convert_type(buf[pl.ds(i * L, L)],
                                                    jnp.int32)
                    match = lax.shift_right_logical(bits, 7) == prefix24
                    d = lax.bitwise_and(bits, jnp.int32(0x7F))
                    plsc.addupdate_scatter(hist, [lane * NB + d], ones_i,
                                           mask=match)
                    return 0
                lax.fori_loop(0, VPC, vbody, 0)
                return carry

            _stream_pass(loss_hbm, s, buf0, buf1, sem0, sem1, chunk_c, 0)

        b3, _r4, _c4 = _select(hist, tmp, tmp2, r3, NB_C)
        results.append(prefix24 * 128 + b3)

    q0, q1 = results
    row_i = jnp.where(lane == 0, jnp.full((L,), q0, jnp.int32),
                      jnp.where(lane == 1, jnp.full((L,), q1, jnp.int32),
                                jnp.zeros((L,), jnp.int32)))
    outbuf_i[...] = row_i
    pltpu.sync_copy(outbuf_i, qbits_hbm.at[wid])


@jax.jit
def kernel(predicted, target, mask):
    pred3 = predicted.reshape(B, H, W)
    tgt3 = target.reshape(B, H, W)
    mask3 = mask.reshape(B, H, W)

    loss, sums = _tc_loss(pred3, tgt3, mask3)

    mesh = plsc.VectorSubcoreMesh(core_axis_name="c", subcore_axis_name="s",
                                  num_cores=NC, num_subcores=NS)
    qbits = pl.kernel(
        _sc_body,
        out_type=jax.ShapeDtypeStruct((NW, L), jnp.int32),
        mesh=mesh,
        compiler_params=pltpu.CompilerParams(needs_layout_passes=False),
        scratch_types=[
            pltpu.VMEM((CH,), jnp.float32),
            pltpu.VMEM((CH,), jnp.float32),
            pltpu.VMEM((CAP + L,), jnp.int32),
            pltpu.VMEM((NB * L,), jnp.int32),
            pltpu.VMEM((NB,), jnp.int32),
            pltpu.VMEM((NB // L,), jnp.int32),
            pltpu.VMEM((L,), jnp.int32),
            pltpu.SemaphoreType.DMA,
            pltpu.SemaphoreType.DMA,
        ],
    )(loss.reshape(B, N))

    q_loss = lax.bitcast_convert_type(qbits[:, :SPT].reshape(B), jnp.float32)
    wmae = jnp.sum(sums[:, 0, 0]) / (B * N)
    return (q_loss, wmae)


# speedup vs baseline: 1.0639x; 1.0639x over previous
import jax
import jax.numpy as jnp
from jax import lax
from jax.experimental import pallas as pl
from jax.experimental.pallas import tpu as pltpu
from jax.experimental.pallas import tpu_sc as plsc

B = 64
H = 512
W = 512
N = H * W
K = 1 + round(0.01 * 99.0 * (N - 1))

NC = 2
NS = 16
NW = NC * NS
SPT = B // NW

NB = 4096
NB_C = 128
L = 16
CH = 16384
RPC = CH // W
NCH = N // CH
VPC = CH // L
UNR = 4
CAP = 16384


def _tc_body(pred_ref, tgt_ref, mask_ref, loss_ref, sums_ref):
    lv = mask_ref[...] * lax.abs(pred_ref[...] - tgt_ref[...])
    loss_ref[...] = lv
    sums_ref[...] = jnp.full((1, 1, 128), jnp.sum(lv), jnp.float32)


def _tc_loss(pred, tgt, mask):
    return pl.pallas_call(
        _tc_body,
        grid=(B,),
        in_specs=[
            pl.BlockSpec((1, H, W), lambda b: (b, 0, 0)),
            pl.BlockSpec((1, H, W), lambda b: (b, 0, 0)),
            pl.BlockSpec((1, H, W), lambda b: (b, 0, 0)),
        ],
        out_specs=[
            pl.BlockSpec((1, H, W), lambda b: (b, 0, 0)),
            pl.BlockSpec((1, 1, 128), lambda b: (b, 0, 0)),
        ],
        out_shape=[
            jax.ShapeDtypeStruct((B, H, W), jnp.float32),
            jax.ShapeDtypeStruct((B, 1, 128), jnp.float32),
        ],
    )(pred, tgt, mask)


def _zero_hist(hist, nb):
    def body(i, _):
        for bank in range(L):
            hist[pl.ds(bank * NB + i * L, L)] = jnp.zeros((L,), jnp.int32)
        return 0
    lax.fori_loop(0, nb // L, body, 0)


def _select(hist, tmp, tmp2, r, nb):
    lane = lax.broadcasted_iota(jnp.int32, (L,), 0)
    zvec = jnp.zeros((L,), jnp.int32)
    ones_v = jnp.ones((L,), jnp.int32)
    nblk = nb // L

    def p1(j, _):
        v = zvec
        for bank in range(L):
            v = v + hist[pl.ds(bank * NB + j * L, L)]
            hist[pl.ds(bank * NB + j * L, L)] = zvec
        tmp[pl.ds(j * L, L)] = v
        return 0

    lax.fori_loop(0, nblk, p1, 0)

    def p2a(j, _):
        t = tmp[pl.ds(j * L, L)]
        tot = jnp.sum(t)
        plsc.store_scatter(tmp2, [jnp.full((L,), j, jnp.int32)],
                           jnp.full((L,), tot, jnp.int32),
                           mask=lane == 0)
        return 0

    lax.fori_loop(0, nblk, p2a, 0)

    zero = jnp.int32(0)
    nv2 = (nblk + L - 1) // L

    def p2b(jj, carry):
        cum, bblk, cumbef = carry
        t = tmp2[pl.ds(jj * L, L)]
        t = jnp.where((jj * L + lane) < nblk, t, zvec)
        cv = plsc.cumsum(t) + cum
        mlt = cv < r
        bblk = bblk + jnp.sum(jnp.where(mlt, ones_v, zvec))
        cumbef = cumbef + jnp.sum(jnp.where(mlt, t, zvec))
        cum = cum + jnp.sum(t)
        return (cum, bblk, cumbef)

    _cum, jb, cumbef = lax.fori_loop(0, nv2, p2b, (zero, zero, zero))

    v = tmp[pl.ds(jb * L, L)]
    cv = plsc.cumsum(v) + cumbef
    mlt = cv < r
    msel = jnp.logical_and(cv >= r, (cv - v) < r)
    loff = jnp.sum(jnp.where(mlt, ones_v, zvec))
    cumbef = cumbef + jnp.sum(jnp.where(mlt, v, zvec))
    cnt = jnp.sum(jnp.where(msel, v, zvec))
    return jb * L + loff, r - cumbef, cnt


def _stream_pass(src_hbm, s, buf0, buf1, sem0, sem1, chunk_fn, init_carry):
    def start(c, buf, sem):
        off = pl.multiple_of(c * RPC, RPC)
        pltpu.async_copy(src_hbm.at[s, pl.ds(off, RPC), :], buf, sem)

    def wait(c, buf, sem):
        off = pl.multiple_of(c * RPC, RPC)
        pltpu.make_async_copy(src_hbm.at[s, pl.ds(off, RPC), :], buf,
                              sem).wait()

    start(0, buf0, sem0)
    start(1, buf1, sem1)

    def body(c2, carry):
        c0 = c2 * 2
        wait(c0, buf0, sem0)
        carry = chunk_fn(buf0, c0, carry)

        @pl.when(c0 + 2 < NCH)
        def _():
            start(c0 + 2, buf0, sem0)

        wait(c0 + 1, buf1, sem1)
        carry = chunk_fn(buf1, c0 + 1, carry)

        @pl.when(c0 + 3 < NCH)
        def _():
            start(c0 + 3, buf1, sem1)

        return carry

    return lax.fori_loop(0, NCH // 2, body, init_carry)


def _buf_vreg(buf, i):
    r = lax.shift_right_logical(i, 5)
    co = lax.shift_left(lax.bitwise_and(i, jnp.int32(31)), 4)
    return lax.bitcast_convert_type(buf[r, pl.ds(co, L)], jnp.int32)


def _sc_body(loss_hbm, qbits_hbm,
             buf0, buf1, cand, hist, tmp, tmp2, outbuf_i, sem0, sem1):
    wid = lax.axis_index("s") * NC + lax.axis_index("c")
    lane = lax.broadcasted_iota(jnp.int32, (L,), 0)
    ones_i = jnp.ones((L,), jnp.int32)
    lane_base = lane * NB

    _zero_hist(hist, NB)

    results = []
    for local in range(SPT):
        s = wid * SPT + local

        def chunk_a(buf, c, carry):
            def vbody(i, _):
                for u in range(UNR):
                    bits = _buf_vreg(buf, i * UNR + u)
                    d = lax.shift_right_logical(bits, 19)
                    plsc.addupdate_scatter(hist, [lane_base + d], ones_i)
                return 0
            lax.fori_loop(0, VPC // UNR, vbody, 0)
            return carry

        _stream_pass(loss_hbm, s, buf0, buf1, sem0, sem1, chunk_a, 0)
        b1, r2, cnt1 = _select(hist, tmp, tmp2, jnp.int32(K), NB)
        docap = cnt1 <= CAP

        def fast_path():
            def chunk_b(buf, c, cnt_splat):
                def vbody(i, cnt_splat):
                    for u in range(UNR):
                        bits = _buf_vreg(buf, i * UNR + u)
                        match = lax.shift_right_logical(bits, 19) == b1
                        mi = match.astype(jnp.int32)
                        pos = cnt_splat + plsc.cumsum(mi) - 1
                        plsc.store_scatter(cand, [pos], bits, mask=match)
                        cnt_splat = cnt_splat + \
                            plsc.all_reduce_population_count(match)
                    return cnt_splat
                return lax.fori_loop(0, VPC // UNR, vbody, cnt_splat)

            _stream_pass(loss_hbm, s, buf0, buf1, sem0, sem1, chunk_b,
                         jnp.zeros((L,), jnp.int32))
            nv = (cnt1 + L - 1) // L

            def hist2(i, _):
                bits = cand[pl.ds(i * L, L)]
                inb = (i * L + lane) < cnt1
                d = lax.bitwise_and(lax.shift_right_logical(bits, 7),
                                    jnp.int32(0xFFF))
                plsc.addupdate_scatter(hist, [lane_base + d], ones_i,
                                       mask=inb)
                return 0

            lax.fori_loop(0, nv, hist2, 0)
            b2, r3, _c2 = _select(hist, tmp, tmp2, r2, NB)
            pref = b1 * 4096 + b2

            def hist3(i, _):
                bits = cand[pl.ds(i * L, L)]
                inb = (i * L + lane) < cnt1
                match = jnp.logical_and(
                    lax.shift_right_logical(bits, 7) == pref, inb)
                d = lax.bitwise_and(bits, jnp.int32(0x7F))
                plsc.addupdate_scatter(hist, [lane_base + d], ones_i,
                                       mask=match)
                return 0

            lax.fori_loop(0, nv, hist3, 0)
            b3, _r4, _c4 = _select(hist, tmp, tmp2, r3, NB_C)
            return pref * 128 + b3

        def slow_path():
            def chunk_b(buf, c, carry):
                def vbody(i, _):
                    for u in range(UNR):
                        bits = _buf_vreg(buf, i * UNR + u)
                        match = lax.shift_right_logical(bits, 19) == b1
                        d = lax.bitwise_and(lax.shift_right_logical(bits, 7),
                                            jnp.int32(0xFFF))
                        plsc.addupdate_scatter(hist, [lane_base + d], ones_i,
                                               mask=match)
                    return 0
                lax.fori_loop(0, VPC // UNR, vbody, 0)
                return carry

            _stream_pass(loss_hbm, s, buf0, buf1, sem0, sem1, chunk_b, 0)
            b2, r3, _c2 = _select(hist, tmp, tmp2, r2, NB)
            pref = b1 * 4096 + b2

            def chunk_c(buf, c, carry):
                def vbody(i, _):
                    for u in range(UNR):
                        bits = _buf_vreg(buf, i * UNR + u)
                        match = lax.shift_right_logical(bits, 7) == pref
                        d = lax.bitwise_and(bits, jnp.int32(0x7F))
                        plsc.addupdate_scatter(hist, [lane_base + d], ones_i,
                                               mask=match)
                    return 0
                lax.fori_loop(0, VPC // UNR, vbody, 0)
                return carry

            _stream_pass(loss_hbm, s, buf0, buf1, sem0, sem1, chunk_c, 0)
            b3, _r4, _c4 = _select(hist, tmp, tmp2, r3, NB_C)
            return pref * 128 + b3

        qb = lax.cond(docap, fast_path, slow_path)
        results.append(qb)

    q0, q1 = results
    row_i = jnp.where(lane == 0, jnp.full((L,), q0, jnp.int32),
                      jnp.where(lane == 1, jnp.full((L,), q1, jnp.int32),
                                jnp.zeros((L,), jnp.int32)))
    outbuf_i[...] = row_i
    pltpu.sync_copy(outbuf_i, qbits_hbm.at[wid])


@jax.jit
def kernel(predicted, target, mask):
    pred3 = predicted.reshape(B, H, W)
    tgt3 = target.reshape(B, H, W)
    mask3 = mask.reshape(B, H, W)

    loss, sums = _tc_loss(pred3, tgt3, mask3)

    mesh = plsc.VectorSubcoreMesh(core_axis_name="c", subcore_axis_name="s",
                                  num_cores=NC, num_subcores=NS)
    qbits = pl.kernel(
        _sc_body,
        out_type=jax.ShapeDtypeStruct((NW, L), jnp.int32),
        mesh=mesh,
        compiler_params=pltpu.CompilerParams(needs_layout_passes=False),
        scratch_types=[
            pltpu.VMEM((RPC, W), jnp.float32),
            pltpu.VMEM((RPC, W), jnp.float32),
            pltpu.VMEM((CAP + L,), jnp.int32),
            pltpu.VMEM((NB * L,), jnp.int32),
            pltpu.VMEM((NB,), jnp.int32),
            pltpu.VMEM((NB // L,), jnp.int32),
            pltpu.VMEM((L,), jnp.int32),
            pltpu.SemaphoreType.DMA,
            pltpu.SemaphoreType.DMA,
        ],
    )(loss)

    q_loss = lax.bitcast_convert_type(qbits[:, :SPT].reshape(B), jnp.float32)
    wmae = jnp.sum(sums[:, 0, 0]) / (B * N)
    return (q_loss, wmae)
